# speedup vs baseline: 1.9338x; 1.9338x over previous
import jax
import jax.numpy as jnp
from jax.experimental import pallas as pl
from jax.experimental.pallas import tpu as pltpu

_ROWS = 512


def _body(lab_ref, seg_ref, x_ref, pos_ref, o_ref):
    l = lab_ref[0, 0, :][:, None]
    s0 = seg_ref[0, :][None, :]
    s1 = seg_ref[1, :][None, :]
    s2 = seg_ref[2, :][None, :]
    seg = jnp.where(l == 0, s0, jnp.where(l == 1, s1, s2))
    o_ref[...] = x_ref[...] + pos_ref[...] + seg


def kernel(x, segment_label, seg_table, pos_table):
    B, S, D = x.shape
    BS = B * S
    rows = _ROWS
    n_blocks = BS // rows
    blocks_per_batch = S // rows

    x2 = x.reshape(BS, D)
    lab = segment_label.astype(jnp.int32).reshape(n_blocks, 1, rows)
    seg_p = jnp.zeros((8, D), seg_table.dtype).at[:3, :].set(seg_table)

    out = pl.pallas_call(
        _body,
        grid=(n_blocks,),
        in_specs=[
            pl.BlockSpec((1, 1, rows), lambda i: (i, 0, 0)),
            pl.BlockSpec((8, D), lambda i: (0, 0)),
            pl.BlockSpec((rows, D), lambda i: (i, 0)),
            pl.BlockSpec((rows, D), lambda i: (i % blocks_per_batch, 0)),
        ],
        out_specs=pl.BlockSpec((rows, D), lambda i: (i, 0)),
        out_shape=jax.ShapeDtypeStruct((BS, D), x.dtype),
    )(lab, seg_p, x2, pos_table)
    return out.reshape(B, S, D)


# speedup vs baseline: 2.2582x; 1.1677x over previous
import jax
import jax.numpy as jnp
from jax.experimental import pallas as pl
from jax.experimental.pallas import tpu as pltpu

_ROWS = 512


def _body(lab_ref, seg_ref, x_ref, pos_ref, o_ref):
    l = lab_ref[0, 0, :][:, None]
    s0 = seg_ref[0, :][None, :]
    s1 = seg_ref[1, :][None, :]
    s2 = seg_ref[2, :][None, :]
    seg = jnp.where(l == 0, s0, jnp.where(l == 1, s1, s2))
    o_ref[...] = x_ref[...] + pos_ref[...] + seg


def kernel(x, segment_label, seg_table, pos_table):
    B, S, D = x.shape
    BS = B * S
    rows = _ROWS
    n_blocks = BS // rows
    blocks_per_batch = S // rows

    x2 = x.reshape(BS, D)
    lab = segment_label.astype(jnp.int32).reshape(n_blocks, 1, rows)
    seg_p = jnp.zeros((8, D), seg_table.dtype).at[:3, :].set(seg_table)

    out = pl.pallas_call(
        _body,
        grid=(blocks_per_batch, B),
        in_specs=[
            pl.BlockSpec((1, 1, rows), lambda j, b: (b * blocks_per_batch + j, 0, 0)),
            pl.BlockSpec((8, D), lambda j, b: (0, 0)),
            pl.BlockSpec((rows, D), lambda j, b: (b * blocks_per_batch + j, 0)),
            pl.BlockSpec((rows, D), lambda j, b: (j, 0)),
        ],
        out_specs=pl.BlockSpec((rows, D), lambda j, b: (b * blocks_per_batch + j, 0)),
        out_shape=jax.ShapeDtypeStruct((BS, D), x.dtype),
    )(lab, seg_p, x2, pos_table)
    return out.reshape(B, S, D)


# speedup vs baseline: 2.3883x; 1.0576x over previous
import jax
import jax.numpy as jnp
from jax.experimental import pallas as pl
from jax.experimental.pallas import tpu as pltpu

_ROWS = 1024


def _body(lab_ref, seg_ref, x_ref, pos_ref, o_ref):
    l = lab_ref[0, 0, :][:, None]
    s0 = seg_ref[0, :][None, :]
    s1 = seg_ref[1, :][None, :]
    s2 = seg_ref[2, :][None, :]
    seg = jnp.where(l == 0, s0, jnp.where(l == 1, s1, s2))
    o_ref[...] = x_ref[...] + pos_ref[...] + seg


def kernel(x, segment_label, seg_table, pos_table):
    B, S, D = x.shape
    BS = B * S
    rows = _ROWS
    n_blocks = BS // rows
    blocks_per_batch = S // rows

    x2 = x.reshape(BS, D)
    lab = segment_label.astype(jnp.int32).reshape(n_blocks, 1, rows)
    seg_p = jnp.zeros((8, D), seg_table.dtype).at[:3, :].set(seg_table)

    out = pl.pallas_call(
        _body,
        grid=(blocks_per_batch, B),
        in_specs=[
            pl.BlockSpec((1, 1, rows), lambda j, b: (b * blocks_per_batch + j, 0, 0)),
            pl.BlockSpec((8, D), lambda j, b: (0, 0)),
            pl.BlockSpec((rows, D), lambda j, b: (b * blocks_per_batch + j, 0)),
            pl.BlockSpec((rows, D), lambda j, b: (j, 0)),
        ],
        out_specs=pl.BlockSpec((rows, D), lambda j, b: (b * blocks_per_batch + j, 0)),
        out_shape=jax.ShapeDtypeStruct((BS, D), x.dtype),
    )(lab, seg_p, x2, pos_table)
    return out.reshape(B, S, D)


# speedup vs baseline: 2.6510x; 1.1100x over previous
import jax
import jax.numpy as jnp
from jax.experimental import pallas as pl
from jax.experimental.pallas import tpu as pltpu

_ROWS = 2048


def _body(lab_ref, seg_ref, x_ref, pos_ref, o_ref):
    l = lab_ref[0, 0, :][:, None]
    s0 = seg_ref[0, :][None, :]
    s1 = seg_ref[1, :][None, :]
    s2 = seg_ref[2, :][None, :]
    seg = jnp.where(l == 0, s0, jnp.where(l == 1, s1, s2))
    o_ref[...] = x_ref[...] + pos_ref[...] + seg


def kernel(x, segment_label, seg_table, pos_table):
    B, S, D = x.shape
    BS = B * S
    rows = _ROWS
    n_blocks = BS // rows
    blocks_per_batch = S // rows

    x2 = x.reshape(BS, D)
    lab = segment_label.astype(jnp.int32).reshape(n_blocks, 1, rows)
    seg_p = jnp.zeros((8, D), seg_table.dtype).at[:3, :].set(seg_table)

    out = pl.pallas_call(
        _body,
        grid=(blocks_per_batch, B),
        in_specs=[
            pl.BlockSpec((1, 1, rows), lambda j, b: (b * blocks_per_batch + j, 0, 0)),
            pl.BlockSpec((8, D), lambda j, b: (0, 0)),
            pl.BlockSpec((rows, D), lambda j, b: (b * blocks_per_batch + j, 0)),
            pl.BlockSpec((rows, D), lambda j, b: (j, 0)),
        ],
        out_specs=pl.BlockSpec((rows, D), lambda j, b: (b * blocks_per_batch + j, 0)),
        out_shape=jax.ShapeDtypeStruct((BS, D), x.dtype),
    )(lab, seg_p, x2, pos_table)
    return out.reshape(B, S, D)
